# baseline (speedup 1.0000x reference)
import functools

import jax
import jax.numpy as jnp
from jax import lax
from jax.experimental import pallas as pl
from jax.experimental.pallas import tpu as pltpu
from jax.experimental.pallas import tpu_sc as plsc

D_MODEL = 64
NUM_CONT = 16
VOCAB = 1000000

_NC = 2
_NS = 16
_NW = _NC * _NS

_CHUNK = 256
_SUBS = _CHUNK // 128
_IDX_TILE = 1024

_ABLK = 4096
_PBLK = _ABLK // 2
_ASH = _ABLK.bit_length() - 1
_NBLK = -(-VOCAB // _ABLK)
_PACKED_ROWS = _NBLK * _PBLK


def _tc_pair_transpose(tblp):
    def body(x_ref, o_ref):
        a = x_ref[:, 0:_PBLK]
        b = x_ref[:, _PBLK:_ABLK]
        o_ref[...] = jnp.concatenate([a.T, b.T], axis=1)

    return pl.pallas_call(
        body,
        grid=(_NBLK,),
        in_specs=[pl.BlockSpec((D_MODEL, _ABLK), lambda i: (0, i))],
        out_specs=pl.BlockSpec((_PBLK, 2 * D_MODEL), lambda i: (i, 0)),
        out_shape=jax.ShapeDtypeStruct((_PACKED_ROWS, 2 * D_MODEL),
                                       jnp.float32),
    )(tblp)


def _sc_gather(tbl2, idx2d, n_l, n_b):
    n = n_l * n_b
    per_w = n // _NW
    idx_rows_per_w = per_w // 128
    chunks_per_w = per_w // _CHUNK
    nh = chunks_per_w // 2
    cpt = _IDX_TILE // _CHUNK
    mesh = plsc.VectorSubcoreMesh(core_axis_name="c", subcore_axis_name="s")

    @functools.partial(
        pl.kernel,
        out_type=jax.ShapeDtypeStruct((n_l, n_b, 2 * D_MODEL), jnp.float32),
        mesh=mesh,
        scratch_types=[
            pltpu.VMEM((8, 128), jnp.int32),
            pltpu.VMEM((_SUBS, 128), jnp.int32),
            pltpu.VMEM((_SUBS, 128), jnp.int32),
            pltpu.VMEM((_CHUNK, 2 * D_MODEL), jnp.float32),
            pltpu.VMEM((_CHUNK, 2 * D_MODEL), jnp.float32),
            pltpu.SemaphoreType.DMA,
            pltpu.SemaphoreType.DMA,
            pltpu.SemaphoreType.DMA,
            pltpu.SemaphoreType.DMA,
        ],
        compiler_params=pltpu.CompilerParams(needs_layout_passes=False),
    )
    def gather_kernel(tbl_hbm, idx_hbm, out_hbm, idx_v, p0, p1,
                      rows0, rows1, sg0, sg1, so0, so1):
        wid = lax.axis_index("s") * _NC + lax.axis_index("c")
        base_n = wid * per_w
        base_row = wid * idx_rows_per_w

        def stage_tile(t):
            pltpu.sync_copy(idx_hbm.at[pl.ds(base_row + t * 8, 8)], idx_v)

        def prep(c, p_buf):
            m = lax.rem(c, cpt)
            for g in range(_CHUNK // 16):
                row = 2 * m + g // 8
                col = (g % 8) * 16
                iv = idx_v[row, pl.ds(col, 16)]
                pv = ((iv >> _ASH) << (_ASH - 1)) + (iv & (_PBLK - 1))
                p_buf[g // 8, pl.ds((g % 8) * 16, 16)] = pv

        def fire_gather(p_buf, rows_buf, sem):
            for j in range(_SUBS):
                pltpu.async_copy(
                    tbl_hbm.at[p_buf.at[j]],
                    rows_buf.at[pl.ds(j * 128, 128)],
                    sem,
                )

        def wait_gather(p_buf, rows_buf, sem):
            for j in range(_SUBS):
                pltpu.make_async_copy(
                    tbl_hbm.at[p_buf.at[j]],
                    rows_buf.at[pl.ds(j * 128, 128)],
                    sem,
                ).wait()

        def fire_out(c, rows_buf, sem):
            n0 = base_n + c * _CHUNK
            l = n0 // n_b
            b0 = lax.rem(n0, n_b)
            pltpu.async_copy(rows_buf, out_hbm.at[l, pl.ds(b0, _CHUNK)], sem)

        def drain_out(rows_buf, sem):
            pltpu.make_async_copy(
                rows_buf, out_hbm.at[0, pl.ds(0, _CHUNK)], sem).wait()

        stage_tile(0)
        prep(0, p0)
        fire_gather(p0, rows0, sg0)

        def body(h, carry):
            c0 = 2 * h
            c1 = c0 + 1
            c2 = c0 + 2
            prep(c1, p1)

            @pl.when(h >= 1)
            def _():
                drain_out(rows1, so1)

            fire_gather(p1, rows1, sg1)

            wait_gather(p0, rows0, sg0)
            fire_out(c0, rows0, so0)
            wait_gather(p1, rows1, sg1)
            fire_out(c1, rows1, so1)

            @pl.when(h < nh - 1)
            def _():
                @pl.when(lax.rem(c2, cpt) == 0)
                def _():
                    stage_tile(c2 // cpt)

                prep(c2, p0)
                drain_out(rows0, so0)
                fire_gather(p0, rows0, sg0)

            return carry

        lax.fori_loop(0, nh, body, 0)
        drain_out(rows0, so0)
        drain_out(rows1, so1)

    return gather_kernel(tbl2, idx2d)


def _tc_fuse(gathered, idxp3, contp, wt, bias2d, n_l, n_b):
    blk = 1024

    def body(g_ref, i_ref, c_ref, w_ref, b_ref, o_ref):
        gt = g_ref[0].T
        iv = i_ref[0]
        hi_bit = (iv >> (_ASH - 1)) & 1
        sel = jnp.where(
            jnp.broadcast_to(hi_bit != 0, (D_MODEL, blk)),
            gt[D_MODEL:2 * D_MODEL],
            gt[0:D_MODEL],
        )
        proj = lax.dot_general(
            w_ref[...], c_ref[0],
            (((0,), (0,)), ((), ())),
            preferred_element_type=jnp.float32,
        )
        o_ref[0] = sel + proj + b_ref[...]

    return pl.pallas_call(
        body,
        grid=(n_l, n_b // blk),
        in_specs=[
            pl.BlockSpec((1, blk, 2 * D_MODEL), lambda l, j: (l, j, 0)),
            pl.BlockSpec((1, 1, blk), lambda l, j: (l, 0, j)),
            pl.BlockSpec((1, NUM_CONT, blk), lambda l, j: (l, 0, j)),
            pl.BlockSpec((NUM_CONT, D_MODEL), lambda l, j: (0, 0)),
            pl.BlockSpec((D_MODEL, 1), lambda l, j: (0, 0)),
        ],
        out_specs=pl.BlockSpec((1, D_MODEL, blk), lambda l, j: (l, 0, j)),
        out_shape=jax.ShapeDtypeStruct((n_l, D_MODEL, n_b), jnp.float32),
    )(gathered, idxp3, contp, wt, bias2d)


def kernel(binary_data, continuous_data, emb_table, lin_w, lin_b):
    b, l = binary_data.shape
    n = b * l

    tblp = emb_table.T
    idxp = binary_data.T
    contp = jnp.transpose(continuous_data, (1, 2, 0))
    wt = lin_w.T
    bias2d = lin_b.reshape(D_MODEL, 1)

    idx2d = idxp.reshape(n // 128, 128)
    idxp3 = idxp.reshape(l, 1, b)

    tbl2 = _tc_pair_transpose(tblp)
    gathered = _sc_gather(tbl2, idx2d, l, b)
    outp = _tc_fuse(gathered, idxp3, contp, wt, bias2d, l, b)

    return jnp.transpose(outp, (2, 0, 1))


# speedup vs baseline: 1.3900x; 1.3900x over previous
import functools

import jax
import jax.numpy as jnp
from jax import lax
from jax.experimental import pallas as pl
from jax.experimental.pallas import tpu as pltpu
from jax.experimental.pallas import tpu_sc as plsc

D_MODEL = 64
NUM_CONT = 16
VOCAB = 1000000

_NC = 2
_NS = 16
_NW = _NC * _NS

_CHUNK = 256
_SUBS = _CHUNK // 128
_IDX_TILE = 1024

_ABLK = 8192
_PBLK = _ABLK // 2
_ASH = _ABLK.bit_length() - 1
_NBLK = -(-VOCAB // _ABLK)
_PACKED_ROWS = _NBLK * _PBLK


def _tc_pair_transpose(tblp):
    def body(x_ref, o_ref):
        a = x_ref[:, 0:_PBLK]
        b = x_ref[:, _PBLK:_ABLK]
        o_ref[...] = jnp.concatenate([a.T, b.T], axis=1)

    return pl.pallas_call(
        body,
        grid=(_NBLK,),
        in_specs=[pl.BlockSpec((D_MODEL, _ABLK), lambda i: (0, i))],
        out_specs=pl.BlockSpec((_PBLK, 2 * D_MODEL), lambda i: (i, 0)),
        out_shape=jax.ShapeDtypeStruct((_PACKED_ROWS, 2 * D_MODEL),
                                       jnp.float32),
    )(tblp)


def _sc_gather(tbl2, idx2d, n_l, n_b):
    n = n_l * n_b
    per_w = n // _NW
    idx_rows_per_w = per_w // 128
    chunks_per_w = per_w // _CHUNK
    nh = chunks_per_w // 2
    cpt = _IDX_TILE // _CHUNK
    mesh = plsc.VectorSubcoreMesh(core_axis_name="c", subcore_axis_name="s")

    @functools.partial(
        pl.kernel,
        out_type=jax.ShapeDtypeStruct((n_l, n_b, 2 * D_MODEL), jnp.float32),
        mesh=mesh,
        scratch_types=[
            pltpu.VMEM((8, 128), jnp.int32),
            pltpu.VMEM((_SUBS, 128), jnp.int32),
            pltpu.VMEM((_SUBS, 128), jnp.int32),
            pltpu.VMEM((_CHUNK, 2 * D_MODEL), jnp.float32),
            pltpu.VMEM((_CHUNK, 2 * D_MODEL), jnp.float32),
            pltpu.SemaphoreType.DMA,
            pltpu.SemaphoreType.DMA,
            pltpu.SemaphoreType.DMA,
            pltpu.SemaphoreType.DMA,
        ],
        compiler_params=pltpu.CompilerParams(needs_layout_passes=False),
    )
    def gather_kernel(tbl_hbm, idx_hbm, out_hbm, idx_v, p0, p1,
                      rows0, rows1, sg0, sg1, so0, so1):
        wid = lax.axis_index("s") * _NC + lax.axis_index("c")
        base_n = wid * per_w
        base_row = wid * idx_rows_per_w

        def stage_tile(t):
            pltpu.sync_copy(idx_hbm.at[pl.ds(base_row + t * 8, 8)], idx_v)

        def prep(c, p_buf):
            m = lax.rem(c, cpt)
            for g in range(_CHUNK // 16):
                row = 2 * m + g // 8
                col = (g % 8) * 16
                iv = idx_v[row, pl.ds(col, 16)]
                pv = ((iv >> _ASH) << (_ASH - 1)) + (iv & (_PBLK - 1))
                p_buf[g // 8, pl.ds((g % 8) * 16, 16)] = pv

        def fire_gather(p_buf, rows_buf, sem):
            for j in range(_SUBS):
                pltpu.async_copy(
                    tbl_hbm.at[p_buf.at[j]],
                    rows_buf.at[pl.ds(j * 128, 128)],
                    sem,
                )

        def wait_gather(p_buf, rows_buf, sem):
            for j in range(_SUBS):
                pltpu.make_async_copy(
                    tbl_hbm.at[p_buf.at[j]],
                    rows_buf.at[pl.ds(j * 128, 128)],
                    sem,
                ).wait()

        def fire_out(c, rows_buf, sem):
            n0 = base_n + c * _CHUNK
            l = n0 // n_b
            b0 = lax.rem(n0, n_b)
            pltpu.async_copy(rows_buf, out_hbm.at[l, pl.ds(b0, _CHUNK)], sem)

        def drain_out(rows_buf, sem):
            pltpu.make_async_copy(
                rows_buf, out_hbm.at[0, pl.ds(0, _CHUNK)], sem).wait()

        stage_tile(0)
        prep(0, p0)
        fire_gather(p0, rows0, sg0)

        def body(h, carry):
            c0 = 2 * h
            c1 = c0 + 1
            c2 = c0 + 2
            prep(c1, p1)

            @pl.when(h >= 1)
            def _():
                drain_out(rows1, so1)

            fire_gather(p1, rows1, sg1)

            wait_gather(p0, rows0, sg0)
            fire_out(c0, rows0, so0)
            wait_gather(p1, rows1, sg1)
            fire_out(c1, rows1, so1)

            @pl.when(h < nh - 1)
            def _():
                @pl.when(lax.rem(c2, cpt) == 0)
                def _():
                    stage_tile(c2 // cpt)

                prep(c2, p0)
                drain_out(rows0, so0)
                fire_gather(p0, rows0, sg0)

            return carry

        lax.fori_loop(0, nh, body, 0)
        drain_out(rows0, so0)
        drain_out(rows1, so1)

    return gather_kernel(tbl2, idx2d)


def _tc_fuse(gathered, idxp3, contp, wt, bias2d, n_l, n_b):
    blk = 4096

    def body(g_ref, i_ref, c_ref, w_ref, b_ref, o_ref):
        gt = g_ref[0].T
        iv = i_ref[0]
        hi_bit = (iv >> (_ASH - 1)) & 1
        sel = jnp.where(
            jnp.broadcast_to(hi_bit != 0, (D_MODEL, blk)),
            gt[D_MODEL:2 * D_MODEL],
            gt[0:D_MODEL],
        )
        proj = lax.dot_general(
            w_ref[...], c_ref[0],
            (((0,), (0,)), ((), ())),
            preferred_element_type=jnp.float32,
        )
        o_ref[0] = sel + proj + b_ref[...]

    return pl.pallas_call(
        body,
        grid=(n_l, n_b // blk),
        in_specs=[
            pl.BlockSpec((1, blk, 2 * D_MODEL), lambda l, j: (l, j, 0)),
            pl.BlockSpec((1, 1, blk), lambda l, j: (l, 0, j)),
            pl.BlockSpec((1, NUM_CONT, blk), lambda l, j: (l, 0, j)),
            pl.BlockSpec((NUM_CONT, D_MODEL), lambda l, j: (0, 0)),
            pl.BlockSpec((D_MODEL, 1), lambda l, j: (0, 0)),
        ],
        out_specs=pl.BlockSpec((1, D_MODEL, blk), lambda l, j: (l, 0, j)),
        out_shape=jax.ShapeDtypeStruct((n_l, D_MODEL, n_b), jnp.float32),
    )(gathered, idxp3, contp, wt, bias2d)


def kernel(binary_data, continuous_data, emb_table, lin_w, lin_b):
    b, l = binary_data.shape
    n = b * l

    tblp = emb_table.T
    idxp = binary_data.T
    contp = jnp.transpose(continuous_data, (1, 2, 0))
    wt = lin_w.T
    bias2d = lin_b.reshape(D_MODEL, 1)

    idx2d = idxp.reshape(n // 128, 128)
    idxp3 = idxp.reshape(l, 1, b)

    tbl2 = _tc_pair_transpose(tblp)
    gathered = _sc_gather(tbl2, idx2d, l, b)
    outp = _tc_fuse(gathered, idxp3, contp, wt, bias2d, l, b)

    return jnp.transpose(outp, (2, 0, 1))


# speedup vs baseline: 1.5623x; 1.1239x over previous
import functools

import jax
import jax.numpy as jnp
from jax import lax
from jax.experimental import pallas as pl
from jax.experimental.pallas import tpu as pltpu
from jax.experimental.pallas import tpu_sc as plsc

D_MODEL = 64
NUM_CONT = 16
VOCAB = 1000000

_NC = 2
_NS = 16
_NW = _NC * _NS

_CHUNK = 256
_SUBS = _CHUNK // 128
_IDX_TILE = 1024

_ABLK = 16384
_PBLK = _ABLK // 2
_ASH = _ABLK.bit_length() - 1
_NBLK = -(-VOCAB // _ABLK)
_PACKED_ROWS = _NBLK * _PBLK


def _tc_pair_transpose(tblp):
    def body(x_ref, o_ref):
        a = x_ref[:, 0:_PBLK]
        b = x_ref[:, _PBLK:_ABLK]
        o_ref[...] = jnp.concatenate([a.T, b.T], axis=1)

    return pl.pallas_call(
        body,
        grid=(_NBLK,),
        in_specs=[pl.BlockSpec((D_MODEL, _ABLK), lambda i: (0, i))],
        out_specs=pl.BlockSpec((_PBLK, 2 * D_MODEL), lambda i: (i, 0)),
        out_shape=jax.ShapeDtypeStruct((_PACKED_ROWS, 2 * D_MODEL),
                                       jnp.float32),
    )(tblp)


def _sc_gather(tbl2, idx2d, n_l, n_b):
    n = n_l * n_b
    per_w = n // _NW
    idx_rows_per_w = per_w // 128
    chunks_per_w = per_w // _CHUNK
    nh = chunks_per_w // 2
    cpt = _IDX_TILE // _CHUNK
    mesh = plsc.VectorSubcoreMesh(core_axis_name="c", subcore_axis_name="s")

    @functools.partial(
        pl.kernel,
        out_type=jax.ShapeDtypeStruct((n_l, n_b, 2 * D_MODEL), jnp.float32),
        mesh=mesh,
        scratch_types=[
            pltpu.VMEM((8, 128), jnp.int32),
            pltpu.VMEM((_SUBS, 128), jnp.int32),
            pltpu.VMEM((_SUBS, 128), jnp.int32),
            pltpu.VMEM((_CHUNK, 2 * D_MODEL), jnp.float32),
            pltpu.VMEM((_CHUNK, 2 * D_MODEL), jnp.float32),
            pltpu.SemaphoreType.DMA,
            pltpu.SemaphoreType.DMA,
            pltpu.SemaphoreType.DMA,
            pltpu.SemaphoreType.DMA,
        ],
        compiler_params=pltpu.CompilerParams(needs_layout_passes=False),
    )
    def gather_kernel(tbl_hbm, idx_hbm, out_hbm, idx_v, p0, p1,
                      rows0, rows1, sg0, sg1, so0, so1):
        wid = lax.axis_index("s") * _NC + lax.axis_index("c")
        base_n = wid * per_w
        base_row = wid * idx_rows_per_w

        def stage_tile(t):
            pltpu.sync_copy(idx_hbm.at[pl.ds(base_row + t * 8, 8)], idx_v)

        def prep(c, p_buf):
            m = lax.rem(c, cpt)
            for g in range(_CHUNK // 16):
                row = 2 * m + g // 8
                col = (g % 8) * 16
                iv = idx_v[row, pl.ds(col, 16)]
                pv = ((iv >> _ASH) << (_ASH - 1)) + (iv & (_PBLK - 1))
                p_buf[g // 8, pl.ds((g % 8) * 16, 16)] = pv

        def fire_gather(p_buf, rows_buf, sem):
            for j in range(_SUBS):
                pltpu.async_copy(
                    tbl_hbm.at[p_buf.at[j]],
                    rows_buf.at[pl.ds(j * 128, 128)],
                    sem,
                )

        def wait_gather(p_buf, rows_buf, sem):
            for j in range(_SUBS):
                pltpu.make_async_copy(
                    tbl_hbm.at[p_buf.at[j]],
                    rows_buf.at[pl.ds(j * 128, 128)],
                    sem,
                ).wait()

        def fire_out(c, rows_buf, sem):
            n0 = base_n + c * _CHUNK
            l = n0 // n_b
            b0 = lax.rem(n0, n_b)
            pltpu.async_copy(rows_buf, out_hbm.at[l, pl.ds(b0, _CHUNK)], sem)

        def drain_out(rows_buf, sem):
            pltpu.make_async_copy(
                rows_buf, out_hbm.at[0, pl.ds(0, _CHUNK)], sem).wait()

        stage_tile(0)
        prep(0, p0)
        fire_gather(p0, rows0, sg0)

        def body(h, carry):
            c0 = 2 * h
            c1 = c0 + 1
            c2 = c0 + 2
            prep(c1, p1)

            @pl.when(h >= 1)
            def _():
                drain_out(rows1, so1)

            fire_gather(p1, rows1, sg1)

            wait_gather(p0, rows0, sg0)
            fire_out(c0, rows0, so0)
            wait_gather(p1, rows1, sg1)
            fire_out(c1, rows1, so1)

            @pl.when(h < nh - 1)
            def _():
                @pl.when(lax.rem(c2, cpt) == 0)
                def _():
                    stage_tile(c2 // cpt)

                prep(c2, p0)
                drain_out(rows0, so0)
                fire_gather(p0, rows0, sg0)

            return carry

        lax.fori_loop(0, nh, body, 0)
        drain_out(rows0, so0)
        drain_out(rows1, so1)

    return gather_kernel(tbl2, idx2d)


def _tc_fuse(gathered, idxp3, contp, wt, bias2d, n_l, n_b):
    blk = 8192

    def body(g_ref, i_ref, c_ref, w_ref, b_ref, o_ref):
        gt = g_ref[0].T
        iv = i_ref[0]
        hi_bit = (iv >> (_ASH - 1)) & 1
        sel = jnp.where(
            jnp.broadcast_to(hi_bit != 0, (D_MODEL, blk)),
            gt[D_MODEL:2 * D_MODEL],
            gt[0:D_MODEL],
        )
        proj = lax.dot_general(
            w_ref[...], c_ref[0],
            (((0,), (0,)), ((), ())),
            preferred_element_type=jnp.float32,
        )
        o_ref[0] = sel + proj + b_ref[...]

    return pl.pallas_call(
        body,
        grid=(n_l, n_b // blk),
        in_specs=[
            pl.BlockSpec((1, blk, 2 * D_MODEL), lambda l, j: (l, j, 0)),
            pl.BlockSpec((1, 1, blk), lambda l, j: (l, 0, j)),
            pl.BlockSpec((1, NUM_CONT, blk), lambda l, j: (l, 0, j)),
            pl.BlockSpec((NUM_CONT, D_MODEL), lambda l, j: (0, 0)),
            pl.BlockSpec((D_MODEL, 1), lambda l, j: (0, 0)),
        ],
        out_specs=pl.BlockSpec((1, D_MODEL, blk), lambda l, j: (l, 0, j)),
        out_shape=jax.ShapeDtypeStruct((n_l, D_MODEL, n_b), jnp.float32),
    )(gathered, idxp3, contp, wt, bias2d)


def kernel(binary_data, continuous_data, emb_table, lin_w, lin_b):
    b, l = binary_data.shape
    n = b * l

    tblp = emb_table.T
    idxp = binary_data.T
    contp = jnp.transpose(continuous_data, (1, 2, 0))
    wt = lin_w.T
    bias2d = lin_b.reshape(D_MODEL, 1)

    idx2d = idxp.reshape(n // 128, 128)
    idxp3 = idxp.reshape(l, 1, b)

    tbl2 = _tc_pair_transpose(tblp)
    gathered = _sc_gather(tbl2, idx2d, l, b)
    outp = _tc_fuse(gathered, idxp3, contp, wt, bias2d, l, b)

    return jnp.transpose(outp, (2, 0, 1))


# speedup vs baseline: 1.6283x; 1.0422x over previous
import functools

import jax
import jax.numpy as jnp
from jax import lax
from jax.experimental import pallas as pl
from jax.experimental.pallas import tpu as pltpu
from jax.experimental.pallas import tpu_sc as plsc

D_MODEL = 64
NUM_CONT = 16
VOCAB = 1000000

_NC = 2
_NS = 16
_NW = _NC * _NS

_CHUNK = 256
_SUBS = _CHUNK // 128
_IDX_TILE = 1024

_ABLK = 32768
_PBLK = _ABLK // 2
_ASH = _ABLK.bit_length() - 1
_NBLK = -(-VOCAB // _ABLK)
_PACKED_ROWS = _NBLK * _PBLK


def _tc_pair_transpose(tblp):
    def body(x_ref, o_ref):
        a = x_ref[:, 0:_PBLK]
        b = x_ref[:, _PBLK:_ABLK]
        o_ref[...] = jnp.concatenate([a.T, b.T], axis=1)

    return pl.pallas_call(
        body,
        grid=(_NBLK,),
        in_specs=[pl.BlockSpec((D_MODEL, _ABLK), lambda i: (0, i))],
        out_specs=pl.BlockSpec((_PBLK, 2 * D_MODEL), lambda i: (i, 0)),
        out_shape=jax.ShapeDtypeStruct((_PACKED_ROWS, 2 * D_MODEL),
                                       jnp.float32),
    )(tblp)


def _sc_gather(tbl2, idx2d, n_l, n_b):
    n = n_l * n_b
    per_w = n // _NW
    idx_rows_per_w = per_w // 128
    chunks_per_w = per_w // _CHUNK
    nh = chunks_per_w // 2
    cpt = _IDX_TILE // _CHUNK
    mesh = plsc.VectorSubcoreMesh(core_axis_name="c", subcore_axis_name="s")

    @functools.partial(
        pl.kernel,
        out_type=jax.ShapeDtypeStruct((n_l, n_b, 2 * D_MODEL), jnp.float32),
        mesh=mesh,
        scratch_types=[
            pltpu.VMEM((8, 128), jnp.int32),
            pltpu.VMEM((_SUBS, 128), jnp.int32),
            pltpu.VMEM((_SUBS, 128), jnp.int32),
            pltpu.VMEM((_CHUNK, 2 * D_MODEL), jnp.float32),
            pltpu.VMEM((_CHUNK, 2 * D_MODEL), jnp.float32),
            pltpu.SemaphoreType.DMA,
            pltpu.SemaphoreType.DMA,
            pltpu.SemaphoreType.DMA,
            pltpu.SemaphoreType.DMA,
        ],
        compiler_params=pltpu.CompilerParams(needs_layout_passes=False),
    )
    def gather_kernel(tbl_hbm, idx_hbm, out_hbm, idx_v, p0, p1,
                      rows0, rows1, sg0, sg1, so0, so1):
        wid = lax.axis_index("s") * _NC + lax.axis_index("c")
        base_n = wid * per_w
        base_row = wid * idx_rows_per_w

        def stage_tile(t):
            pltpu.sync_copy(idx_hbm.at[pl.ds(base_row + t * 8, 8)], idx_v)

        def prep(c, p_buf):
            m = lax.rem(c, cpt)
            for g in range(_CHUNK // 16):
                row = 2 * m + g // 8
                col = (g % 8) * 16
                iv = idx_v[row, pl.ds(col, 16)]
                pv = ((iv >> _ASH) << (_ASH - 1)) + (iv & (_PBLK - 1))
                p_buf[g // 8, pl.ds((g % 8) * 16, 16)] = pv

        def fire_gather(p_buf, rows_buf, sem):
            for j in range(_SUBS):
                pltpu.async_copy(
                    tbl_hbm.at[p_buf.at[j]],
                    rows_buf.at[pl.ds(j * 128, 128)],
                    sem,
                )

        def wait_gather(p_buf, rows_buf, sem):
            for j in range(_SUBS):
                pltpu.make_async_copy(
                    tbl_hbm.at[p_buf.at[j]],
                    rows_buf.at[pl.ds(j * 128, 128)],
                    sem,
                ).wait()

        def fire_out(c, rows_buf, sem):
            n0 = base_n + c * _CHUNK
            l = n0 // n_b
            b0 = lax.rem(n0, n_b)
            pltpu.async_copy(rows_buf, out_hbm.at[l, pl.ds(b0, _CHUNK)], sem)

        def drain_out(rows_buf, sem):
            pltpu.make_async_copy(
                rows_buf, out_hbm.at[0, pl.ds(0, _CHUNK)], sem).wait()

        stage_tile(0)
        prep(0, p0)
        fire_gather(p0, rows0, sg0)

        def body(h, carry):
            c0 = 2 * h
            c1 = c0 + 1
            c2 = c0 + 2
            prep(c1, p1)

            @pl.when(h >= 1)
            def _():
                drain_out(rows1, so1)

            fire_gather(p1, rows1, sg1)

            wait_gather(p0, rows0, sg0)
            fire_out(c0, rows0, so0)
            wait_gather(p1, rows1, sg1)
            fire_out(c1, rows1, so1)

            @pl.when(h < nh - 1)
            def _():
                @pl.when(lax.rem(c2, cpt) == 0)
                def _():
                    stage_tile(c2 // cpt)

                prep(c2, p0)
                drain_out(rows0, so0)
                fire_gather(p0, rows0, sg0)

            return carry

        lax.fori_loop(0, nh, body, 0)
        drain_out(rows0, so0)
        drain_out(rows1, so1)

    return gather_kernel(tbl2, idx2d)


def _tc_fuse(gathered, idxp3, contp, wt, bias2d, n_l, n_b):
    blk = 16384

    def body(g_ref, i_ref, c_ref, w_ref, b_ref, o_ref):
        gt = g_ref[0].T
        iv = i_ref[0]
        hi_bit = (iv >> (_ASH - 1)) & 1
        sel = jnp.where(
            jnp.broadcast_to(hi_bit != 0, (D_MODEL, blk)),
            gt[D_MODEL:2 * D_MODEL],
            gt[0:D_MODEL],
        )
        proj = lax.dot_general(
            w_ref[...], c_ref[0],
            (((0,), (0,)), ((), ())),
            preferred_element_type=jnp.float32,
        )
        o_ref[0] = sel + proj + b_ref[...]

    return pl.pallas_call(
        body,
        grid=(n_l, n_b // blk),
        in_specs=[
            pl.BlockSpec((1, blk, 2 * D_MODEL), lambda l, j: (l, j, 0)),
            pl.BlockSpec((1, 1, blk), lambda l, j: (l, 0, j)),
            pl.BlockSpec((1, NUM_CONT, blk), lambda l, j: (l, 0, j)),
            pl.BlockSpec((NUM_CONT, D_MODEL), lambda l, j: (0, 0)),
            pl.BlockSpec((D_MODEL, 1), lambda l, j: (0, 0)),
        ],
        out_specs=pl.BlockSpec((1, D_MODEL, blk), lambda l, j: (l, 0, j)),
        out_shape=jax.ShapeDtypeStruct((n_l, D_MODEL, n_b), jnp.float32),
    )(gathered, idxp3, contp, wt, bias2d)


def kernel(binary_data, continuous_data, emb_table, lin_w, lin_b):
    b, l = binary_data.shape
    n = b * l

    tblp = emb_table.T
    idxp = binary_data.T
    contp = jnp.transpose(continuous_data, (1, 2, 0))
    wt = lin_w.T
    bias2d = lin_b.reshape(D_MODEL, 1)

    idx2d = idxp.reshape(n // 128, 128)
    idxp3 = idxp.reshape(l, 1, b)

    tbl2 = _tc_pair_transpose(tblp)
    gathered = _sc_gather(tbl2, idx2d, l, b)
    outp = _tc_fuse(gathered, idxp3, contp, wt, bias2d, l, b)

    return jnp.transpose(outp, (2, 0, 1))
